# Initial kernel scaffold; baseline (speedup 1.0000x reference)
import jax
import jax.numpy as jnp
from jax.experimental import pallas as pl


def kernel(hidden_states, gate_w, e_score_correction_bias, w13, w2, shared_w13, shared_w2):
    raise NotImplementedError("write your pallas kernel here")


# speedup vs baseline: 2.8568x; 2.8568x over previous
import functools

import jax
import jax.numpy as jnp
from jax.experimental import pallas as pl
from jax.experimental.pallas import tpu as pltpu

T, D, E, K, FF, NSH = 2048, 1024, 8, 2, 512, 2
RSF = 2.5


def _moe_body(x_ref, gate_ref, bias_ref, w13_ref, w2_ref, sw13_ref, sw2_ref,
              out_ref, meta_ref):
    step = pl.program_id(0)

    @pl.when(step == 0)
    def _gating_and_shared():
        x = x_ref[...]
        logits = jnp.dot(x, gate_ref[...], preferred_element_type=jnp.float32)
        scores = jax.nn.sigmoid(logits)
        sc = scores + bias_ref[...]
        e_iota = jax.lax.broadcasted_iota(jnp.int32, sc.shape, 1)
        m1 = jnp.max(sc, axis=1, keepdims=True)
        i1 = jnp.min(jnp.where(sc == m1, e_iota, E), axis=1, keepdims=True)
        sc2 = jnp.where(e_iota == i1, -jnp.inf, sc)
        m2 = jnp.max(sc2, axis=1, keepdims=True)
        i2 = jnp.min(jnp.where(sc2 == m2, e_iota, E), axis=1, keepdims=True)
        w1 = jnp.sum(jnp.where(e_iota == i1, scores, 0.0), axis=1, keepdims=True)
        w2s = jnp.sum(jnp.where(e_iota == i2, scores, 0.0), axis=1, keepdims=True)
        denom = w1 + w2s + 1e-20
        meta = jnp.concatenate(
            [i1.astype(jnp.float32), i2.astype(jnp.float32),
             w1 / denom * RSF, w2s / denom * RSF,
             jnp.zeros((T, 4), jnp.float32)], axis=1)
        meta_ref[...] = meta
        sgu = jnp.dot(x, sw13_ref[...], preferred_element_type=jnp.float32)
        sg = sgu[:, :FF * NSH]
        su = sgu[:, FF * NSH:]
        sh = jax.nn.silu(sg) * su
        out_ref[...] = jnp.dot(sh, sw2_ref[...], preferred_element_type=jnp.float32)

    @pl.when(step > 0)
    def _routed_expert():
        e = step - 1
        x = x_ref[...]
        gu = jnp.dot(x, w13_ref[0], preferred_element_type=jnp.float32)
        g = gu[:, :FF]
        u = gu[:, FF:]
        h = jax.nn.silu(g) * u
        y = jnp.dot(h, w2_ref[0], preferred_element_type=jnp.float32)
        i1 = meta_ref[:, 0:1]
        i2 = meta_ref[:, 1:2]
        cw1 = meta_ref[:, 2:3]
        cw2 = meta_ref[:, 3:4]
        ef = jnp.float32(1.0) * e
        col = jnp.where(i1 == ef, cw1, 0.0) + jnp.where(i2 == ef, cw2, 0.0)
        out_ref[...] += col * y


@jax.jit
def kernel(hidden_states, gate_w, e_score_correction_bias, w13, w2,
           shared_w13, shared_w2):
    bias2d = e_score_correction_bias.reshape(1, E)
    return pl.pallas_call(
        _moe_body,
        grid=(E + 1,),
        in_specs=[
            pl.BlockSpec((T, D), lambda s: (0, 0)),
            pl.BlockSpec((D, E), lambda s: (0, 0)),
            pl.BlockSpec((1, E), lambda s: (0, 0)),
            pl.BlockSpec((1, D, 2 * FF), lambda s: (jnp.maximum(s - 1, 0), 0, 0)),
            pl.BlockSpec((1, FF, D), lambda s: (jnp.maximum(s - 1, 0), 0, 0)),
            pl.BlockSpec((D, 2 * FF * NSH), lambda s: (0, 0)),
            pl.BlockSpec((FF * NSH, D), lambda s: (0, 0)),
        ],
        out_specs=pl.BlockSpec((T, D), lambda s: (0, 0)),
        out_shape=jax.ShapeDtypeStruct((T, D), jnp.float32),
        scratch_shapes=[pltpu.VMEM((T, 8), jnp.float32)],
    )(hidden_states, gate_w, bias2d, w13, w2, shared_w13, shared_w2)
